# Initial kernel scaffold; baseline (speedup 1.0000x reference)
import jax
import jax.numpy as jnp
from jax.experimental import pallas as pl


def kernel(title_int, body_int, user_int, w_table, c_table):
    raise NotImplementedError("write your pallas kernel here")


# speedup vs baseline: 15.0505x; 15.0505x over previous
import functools

import jax
import jax.numpy as jnp
from jax import lax
from jax.experimental import pallas as pl
from jax.experimental.pallas import tpu as pltpu
from jax.experimental.pallas import tpu_sc as plsc

N = 4096
TL = 50
BL = 200
D = 64
C = 1000
NW = 32
CH = N // NW
BH = 100


def _accum_rows(rows_ref, buf, n, out_ref, e):
    zero = jnp.zeros((16,), jnp.float32)
    unroll = 5

    def body(i, accs):
        a0, a1, a2, a3 = accs
        for u in range(unroll):
            r = i * unroll + u
            a0 = a0 + rows_ref[buf, r, pl.ds(0, 16)]
            a1 = a1 + rows_ref[buf, r, pl.ds(16, 16)]
            a2 = a2 + rows_ref[buf, r, pl.ds(32, 16)]
            a3 = a3 + rows_ref[buf, r, pl.ds(48, 16)]
        return (a0, a1, a2, a3)

    a0, a1, a2, a3 = lax.fori_loop(0, n // unroll, body, (zero, zero, zero, zero))
    out_ref[e, pl.ds(0, 16)] = a0
    out_ref[e, pl.ds(16, 16)] = a1
    out_ref[e, pl.ds(32, 16)] = a2
    out_ref[e, pl.ds(48, 16)] = a3


def _sc_pool_body(w_hbm, title_hbm, body_hbm, tsum_hbm, bsum_hbm,
                  tidx_v, bidx_v, trows, brows, tout, bout, sem0, sem1):
    wid = lax.axis_index("s") * 2 + lax.axis_index("c")
    base = wid * CH

    pltpu.sync_copy(title_hbm.at[pl.ds(base, CH)], tidx_v)
    pltpu.sync_copy(body_hbm.at[pl.ds(2 * base, 2 * CH)], bidx_v)

    sems = (sem0, sem1)

    def start(e, buf):
        sem = sems[buf]
        pltpu.async_copy(w_hbm.at[tidx_v.at[e]], trows.at[buf], sem)
        pltpu.async_copy(w_hbm.at[bidx_v.at[2 * e]], brows.at[buf, pl.ds(0, BH)], sem)
        pltpu.async_copy(w_hbm.at[bidx_v.at[2 * e + 1]], brows.at[buf, pl.ds(BH, BH)], sem)

    def wait(buf):
        sem = sems[buf]
        pltpu.make_async_copy(w_hbm.at[tidx_v.at[0]], trows.at[buf], sem).wait()
        pltpu.make_async_copy(w_hbm.at[bidx_v.at[0]], brows.at[buf, pl.ds(0, BH)], sem).wait()
        pltpu.make_async_copy(w_hbm.at[bidx_v.at[0]], brows.at[buf, pl.ds(BH, BH)], sem).wait()

    def process(e, buf):
        wait(buf)
        _accum_rows(trows, buf, TL, tout, e)
        _accum_rows(brows, buf, BL, bout, e)

    start(0, 0)

    def pair(i, _):
        e0 = 2 * i
        start(e0 + 1, 1)
        process(e0, 0)

        @pl.when(i < CH // 2 - 1)
        def _():
            start(e0 + 2, 0)

        process(e0 + 1, 1)
        return 0

    lax.fori_loop(0, CH // 2, pair, 0)

    pltpu.sync_copy(tout, tsum_hbm.at[pl.ds(base, CH)])
    pltpu.sync_copy(bout, bsum_hbm.at[pl.ds(base, CH)])


_sc_pool = functools.partial(
    pl.kernel,
    out_type=(
        jax.ShapeDtypeStruct((N, D), jnp.float32),
        jax.ShapeDtypeStruct((N, D), jnp.float32),
    ),
    mesh=plsc.VectorSubcoreMesh(core_axis_name="c", subcore_axis_name="s"),
    scratch_types=[
        pltpu.VMEM((CH, TL), jnp.int32),
        pltpu.VMEM((2 * CH, BH), jnp.int32),
        pltpu.VMEM((2, TL, D), jnp.float32),
        pltpu.VMEM((2, BL, D), jnp.float32),
        pltpu.VMEM((CH, D), jnp.float32),
        pltpu.VMEM((CH, D), jnp.float32),
        pltpu.SemaphoreType.DMA,
        pltpu.SemaphoreType.DMA,
    ],
    compiler_params=pltpu.CompilerParams(use_tc_tiling_on_sc=False),
)(_sc_pool_body)


def _head_body(tidx_ref, bidx_ref, ts_ref, bs_ref, c_ref, o_ref):
    tcnt = jnp.sum((tidx_ref[...] > 0).astype(jnp.float32), axis=1, keepdims=True)
    bcnt = jnp.sum((bidx_ref[...] > 0).astype(jnp.float32), axis=1, keepdims=True)
    que = 0.3 * ts_ref[...] / tcnt + 0.7 * bs_ref[...] / bcnt
    sc = lax.dot_general(que, c_ref[...], (((1,), (1,)), ((), ())),
                         preferred_element_type=jnp.float32)
    m = jnp.max(sc, axis=1, keepdims=True)
    e = jnp.exp(sc - m)
    o_ref[...] = e / jnp.sum(e, axis=1, keepdims=True)


_R = 512


def _head(tidx, bidx, tsum, bsum, c_table):
    return pl.pallas_call(
        _head_body,
        out_shape=jax.ShapeDtypeStruct((N, C), jnp.float32),
        grid=(N // _R,),
        in_specs=[
            pl.BlockSpec((_R, TL), lambda i: (i, 0)),
            pl.BlockSpec((_R, BL), lambda i: (i, 0)),
            pl.BlockSpec((_R, D), lambda i: (i, 0)),
            pl.BlockSpec((_R, D), lambda i: (i, 0)),
            pl.BlockSpec((C, D), lambda i: (0, 0)),
        ],
        out_specs=pl.BlockSpec((_R, C), lambda i: (i, 0)),
    )(tidx, bidx, tsum, bsum, c_table)


def kernel(title_int, body_int, user_int, w_table, c_table):
    t = title_int.astype(jnp.int32)
    b = body_int.astype(jnp.int32)
    b2 = b.reshape(2 * N, BH)
    tsum, bsum = _sc_pool(w_table, t, b2)
    return _head(t, b, tsum, bsum, c_table)


# speedup vs baseline: 18.5862x; 1.2349x over previous
import functools

import jax
import jax.numpy as jnp
from jax import lax
from jax.experimental import pallas as pl
from jax.experimental.pallas import tpu as pltpu
from jax.experimental.pallas import tpu_sc as plsc

N = 4096
TL = 50
BL = 200
D = 64
C = 1000
NW = 32
CH = N // NW
BH = 100


def _zero_acc(acc):
    zero = jnp.zeros((16,), jnp.float32)

    def body(e, _):
        acc[e, pl.ds(0, 16)] = zero
        acc[e, pl.ds(16, 16)] = zero
        acc[e, pl.ds(32, 16)] = zero
        acc[e, pl.ds(48, 16)] = zero
        return 0

    lax.fori_loop(0, CH, body, 0)


def _sc_pool_body(w_hbm, titleT_hbm, bodyT_hbm, tsum_hbm, bsum_hbm,
                  tidx_v, bidx_v, acc_t, acc_b, sem_t, sem_b):
    wid = lax.axis_index("s") * 2 + lax.axis_index("c")
    base = wid * CH

    pltpu.sync_copy(titleT_hbm.at[wid], tidx_v)
    pltpu.sync_copy(bodyT_hbm.at[wid], bidx_v)
    _zero_acc(acc_t)
    _zero_acc(acc_b)

    def tpass(k, _):
        pltpu.async_copy(w_hbm.at[tidx_v.at[k]], acc_t, sem_t, add=True)
        return 0

    def bpass(k, _):
        pltpu.async_copy(w_hbm.at[bidx_v.at[k]], acc_b, sem_b, add=True)
        return 0

    lax.fori_loop(0, TL, tpass, 0)
    lax.fori_loop(0, BL, bpass, 0)

    def tdrain(k, _):
        pltpu.make_async_copy(w_hbm.at[tidx_v.at[0]], acc_t, sem_t).wait()
        return 0

    def bdrain(k, _):
        pltpu.make_async_copy(w_hbm.at[bidx_v.at[0]], acc_b, sem_b).wait()
        return 0

    lax.fori_loop(0, TL, tdrain, 0)
    lax.fori_loop(0, BL, bdrain, 0)

    pltpu.sync_copy(acc_t, tsum_hbm.at[pl.ds(base, CH)])
    pltpu.sync_copy(acc_b, bsum_hbm.at[pl.ds(base, CH)])


_sc_pool = functools.partial(
    pl.kernel,
    out_type=(
        jax.ShapeDtypeStruct((N, D), jnp.float32),
        jax.ShapeDtypeStruct((N, D), jnp.float32),
    ),
    mesh=plsc.VectorSubcoreMesh(core_axis_name="c", subcore_axis_name="s"),
    scratch_types=[
        pltpu.VMEM((TL, CH), jnp.int32),
        pltpu.VMEM((BL, CH), jnp.int32),
        pltpu.VMEM((CH, D), jnp.float32),
        pltpu.VMEM((CH, D), jnp.float32),
        pltpu.SemaphoreType.DMA,
        pltpu.SemaphoreType.DMA,
    ],
    compiler_params=pltpu.CompilerParams(use_tc_tiling_on_sc=False),
)(_sc_pool_body)


def _head_body(tidx_ref, bidx_ref, ts_ref, bs_ref, c_ref, o_ref):
    tcnt = jnp.sum((tidx_ref[...] > 0).astype(jnp.float32), axis=1, keepdims=True)
    bcnt = jnp.sum((bidx_ref[...] > 0).astype(jnp.float32), axis=1, keepdims=True)
    que = 0.3 * ts_ref[...] / tcnt + 0.7 * bs_ref[...] / bcnt
    sc = lax.dot_general(que, c_ref[...], (((1,), (1,)), ((), ())),
                         preferred_element_type=jnp.float32)
    m = jnp.max(sc, axis=1, keepdims=True)
    e = jnp.exp(sc - m)
    o_ref[...] = e / jnp.sum(e, axis=1, keepdims=True)


_R = 512


def _head(tidx, bidx, tsum, bsum, c_table):
    return pl.pallas_call(
        _head_body,
        out_shape=jax.ShapeDtypeStruct((N, C), jnp.float32),
        grid=(N // _R,),
        in_specs=[
            pl.BlockSpec((_R, TL), lambda i: (i, 0)),
            pl.BlockSpec((_R, BL), lambda i: (i, 0)),
            pl.BlockSpec((_R, D), lambda i: (i, 0)),
            pl.BlockSpec((_R, D), lambda i: (i, 0)),
            pl.BlockSpec((C, D), lambda i: (0, 0)),
        ],
        out_specs=pl.BlockSpec((_R, C), lambda i: (i, 0)),
    )(tidx, bidx, tsum, bsum, c_table)


def kernel(title_int, body_int, user_int, w_table, c_table):
    t = title_int.astype(jnp.int32)
    b = body_int.astype(jnp.int32)
    tT = t.reshape(NW, CH, TL).transpose(0, 2, 1)
    bT = b.reshape(NW, CH, BL).transpose(0, 2, 1)
    tsum, bsum = _sc_pool(w_table, tT, bT)
    return _head(t, b, tsum, bsum, c_table)
